# baseline (speedup 1.0000x reference)
import functools

import jax
import jax.numpy as jnp
from jax import lax
from jax.experimental import pallas as pl
from jax.experimental.pallas import tpu as pltpu
from jax.experimental.pallas import tpu_sc as plsc

_N, _D = 32768, 512
_B = 16
_SEG = _N // _B
_NC, _NS, _L = 2, 16, 16
_NW = _NC * _NS
_G = _D // _L
_GH = _G // 2
_CHUNK = 64

_S_SC = 8
_S_TC = _B - _S_SC
_SC_BASE = _S_TC * _SEG
_RPW = _S_SC * _SEG // _NW
_NCHUNK = _RPW // _CHUNK
_WPS = _NW // _S_SC


def _sc_partials_body(x_hbm, out_hbm, buf0, buf1, obuf, sem0, sem1):
    c = lax.axis_index("c")
    s = lax.axis_index("s")
    wid = c * _NS + s
    base = _SC_BASE + wid * _RPW
    bufs = (buf0, buf1)
    sems = (sem0, sem1)

    copies = {0: pltpu.async_copy(x_hbm.at[pl.ds(base, _CHUNK)], buf0, sem0)}
    acc = [tuple(jnp.zeros((_L,), jnp.float32) for _ in range(_GH))
           for _ in range(2)]
    for k in range(_NCHUNK):
        copies[k].wait()
        if k + 1 < _NCHUNK:
            copies[k + 1] = pltpu.async_copy(
                x_hbm.at[pl.ds(base + (k + 1) * _CHUNK, _CHUNK)],
                bufs[(k + 1) % 2], sems[(k + 1) % 2])
        cur = bufs[k % 2]

        for h in range(2):
            def row_body(r2, a, cur=cur, h=h):
                for u in range(2):
                    row = 2 * r2 + u
                    a = tuple(
                        a[i] + cur[row, pl.ds((h * _GH + i) * _L, _L)]
                        for i in range(_GH))
                return a

            acc[h] = lax.fori_loop(0, _CHUNK // 2, row_body, acc[h])

    for h in range(2):
        for i in range(_GH):
            obuf[pl.ds((h * _GH + i) * _L, _L)] = acc[h][i]
    pltpu.sync_copy(obuf, out_hbm.at[wid])


_sc_partials = functools.partial(
    pl.kernel,
    out_type=jax.ShapeDtypeStruct((_NW, _D), jnp.float32),
    mesh=plsc.VectorSubcoreMesh(core_axis_name="c", subcore_axis_name="s"),
    scratch_types=[
        pltpu.VMEM((_CHUNK, _D), jnp.float32),
        pltpu.VMEM((_CHUNK, _D), jnp.float32),
        pltpu.VMEM((_D,), jnp.float32),
        pltpu.SemaphoreType.DMA,
        pltpu.SemaphoreType.DMA,
    ],
)(_sc_partials_body)


def _tc_sum_body(x_ref, out_ref):
    out_ref[...] = jnp.sum(x_ref[...], axis=0)[None, None, :]


def kernel(x, batch_lengths):
    tc_sums = pl.pallas_call(
        _tc_sum_body,
        grid=(_S_TC,),
        in_specs=[pl.BlockSpec((_SEG, _D), lambda i: (i, 0))],
        out_specs=pl.BlockSpec((1, 1, _D), lambda i: (i, 0, 0)),
        out_shape=jax.ShapeDtypeStruct((_S_TC, 1, _D), x.dtype),
    )(x)[:, 0, :]
    sc_partials = _sc_partials(x)
    sc_sums = sc_partials.reshape(_S_SC, _WPS, _D).sum(axis=1)
    sums = jnp.concatenate([tc_sums, sc_sums], axis=0)
    return sums / batch_lengths[:, None].astype(x.dtype)
